# baseline (speedup 1.0000x reference)
import functools

import jax
import jax.numpy as jnp
from jax import lax
from jax.experimental import pallas as pl
from jax.experimental.pallas import tpu as pltpu
from jax.experimental.pallas import tpu_sc as plsc

N = 16 * 1024 * 1024
L = 16
CHUNK = 16384
UNROLL = 8
NBUF = 4
SPLIT = 2


def _make_sc_call():
  info = plsc.get_sparse_core_info()
  nw = info.num_cores * info.num_subcores
  per_w = N // nw
  n_chunks = per_w // CHUNK
  n_groups = n_chunks // NBUF
  sub = CHUNK // SPLIT
  cap = CHUNK // L
  vecs = CHUNK // L
  mesh = plsc.VectorSubcoreMesh(core_axis_name="c", subcore_axis_name="s")

  @functools.partial(
      pl.kernel,
      out_type=jax.ShapeDtypeStruct((nw * L,), jnp.int32),
      mesh=mesh,
      compiler_params=pltpu.CompilerParams(needs_layout_passes=False),
      scratch_types=[
          [pltpu.VMEM((CHUNK,), jnp.float32)] * NBUF,
          pltpu.VMEM((CHUNK,), jnp.int32),
          pltpu.VMEM((L,), jnp.int32),
          [pltpu.SemaphoreType.DMA] * NBUF,
      ],
  )
  def sc_kernel(x_hbm, out_hbm, xbs, idxb, flag_v, sems):
    wid = lax.axis_index("s") * info.num_cores + lax.axis_index("c")
    base_w = wid * per_w
    lane = lax.iota(jnp.int32, L)
    ones = jnp.ones((L,), jnp.int32)
    zeros = jnp.zeros((L,), jnp.int32)
    sixteens = jnp.full((L,), L, jnp.int32)

    def compact_and_check(xbuf, base_c, bad):
      def body(t, carry):
        ptr, bad = carry
        v = xbuf[pl.ds(t * L, L)]
        z = v == 0.0
        mi = jnp.where(z, zeros, sixteens)
        idxs = (base_c + t * L) + lane
        plsc.store_scatter(idxb, [ptr], idxs)
        d = plsc.load_gather(idxb, [ptr])
        bad = bad | (d ^ idxs)
        return ptr + mi, bad

      return plsc.parallel_loop(
          0, vecs, unroll=UNROLL, carry=(lane, bad))(body)[1]

    def issue(buf, sem, addr):
      for k in range(SPLIT):
        pltpu.async_copy(
            x_hbm.at[pl.ds(addr + k * sub, sub)], buf.at[pl.ds(k * sub, sub)],
            sem)

    def wait(buf, sem):
      pltpu.make_async_copy(x_hbm.at[pl.ds(0, CHUNK)], buf, sem).wait()

    for b in range(NBUF):
      issue(xbs[b], sems[b], base_w + b * CHUNK)

    def group_body(q, bad):
      c0 = q * NBUF
      for b in range(NBUF):
        base_c = base_w + (c0 + b) * CHUNK
        wait(xbs[b], sems[b])
        bad = compact_and_check(xbs[b], base_c, bad)
        nxt = jnp.minimum(base_c + NBUF * CHUNK, N - CHUNK)
        issue(xbs[b], sems[b], nxt)
      return bad

    bad = lax.fori_loop(0, n_groups, group_body, zeros)
    for b in range(NBUF):
      wait(xbs[b], sems[b])

    flag_v[...] = jnp.where(bad == 0, ones, zeros)
    pltpu.sync_copy(flag_v, out_hbm.at[pl.ds(wid * L, L)])

  return sc_kernel


_sc_call = None


def kernel(x):
  global _sc_call
  if _sc_call is None:
    _sc_call = _make_sc_call()
  flags = _sc_call(x)
  return jnp.all(flags == 1)
